# baseline (speedup 1.0000x reference)
import functools

import jax
import jax.numpy as jnp
from jax import lax
from jax.experimental import pallas as pl
from jax.experimental.pallas import tpu as pltpu
from jax.experimental.pallas import tpu_sc as plsc

NC = 2
NS = 16
NW = NC * NS
LANES = 16
CHUNK = 128
SCH = 64
RING = 4
PAD_ROWS = 16


def _mesh():
    return plsc.VectorSubcoreMesh(core_axis_name="c", subcore_axis_name="s")


def _make_deg_kernel(NP, T):
    nz = NP // NS
    q, r = divmod(T, NW)
    NCHR = (q // 4) * 4

    @functools.partial(
        pl.kernel,
        out_type=jax.ShapeDtypeStruct((NC * NP,), jnp.float32),
        mesh=_mesh(),
        scratch_types=[
            pltpu.VMEM((8, CHUNK), jnp.int32),
            pltpu.VMEM((CHUNK,), jnp.float32),
            pltpu.VMEM((NP // NS,), jnp.float32),
            pltpu.VMEM_SHARED((NP,), jnp.float32),
            pltpu.SemaphoreType.DMA,
            pltpu.SemaphoreType.DMA,
            pltpu.SemaphoreType.DMA,
            pltpu.SemaphoreType.DMA,
        ],
    )
    def deg_kernel(sdx_hbm, out_hbm, idx_v, ones_v, row_v, acc_sh,
                   s0, s1, s2, s3):
        cid = lax.axis_index("c")
        sid = lax.axis_index("s")
        wid = sid * NC + cid
        ld = tuple(idx_v.at[pl.ds(2 * b, 2)] for b in range(4))
        dsti = tuple(idx_v.at[2 * b + 1] for b in range(4))
        sems = (s0, s1, s2, s3)
        base = q * wid + jnp.minimum(wid, r)

        @pl.loop(0, CHUNK // LANES)
        def _(i):
            ones_v[pl.ds(i * LANES, LANES)] = jnp.full((LANES,), 1.0, jnp.float32)

        @pl.loop(0, nz // LANES)
        def _(i):
            row_v[pl.ds(i * LANES, LANES)] = jnp.zeros((LANES,), jnp.float32)

        pltpu.sync_copy(row_v, acc_sh.at[pl.ds(sid * nz, nz)])
        plsc.subcore_barrier()

        for b in range(4):
            pltpu.sync_copy(sdx_hbm.at[base + b], ld[b])
            pltpu.async_copy(ones_v, acc_sh.at[dsti[b]], sems[b], add=True)

        @pl.loop(0, (NCHR - 4) // 4)
        def _(i):
            c = i * 4
            for b in range(4):
                cc = c + b
                pltpu.make_async_copy(ones_v, acc_sh.at[dsti[b]],
                                      sems[b]).wait()
                pltpu.sync_copy(sdx_hbm.at[base + cc + 4], ld[b])
                pltpu.async_copy(ones_v, acc_sh.at[dsti[b]], sems[b],
                                 add=True)

        for b in range(4):
            pltpu.make_async_copy(ones_v, acc_sh.at[dsti[b]], sems[b]).wait()

        for cc in range(NCHR, q):
            pltpu.sync_copy(sdx_hbm.at[base + cc], ld[0])
            pltpu.sync_copy(ones_v, acc_sh.at[dsti[0]], add=True)
        if r:
            @pl.when(wid < r)
            def _():
                pltpu.sync_copy(sdx_hbm.at[base + q], ld[0])
                pltpu.sync_copy(ones_v, acc_sh.at[dsti[0]], add=True)

        plsc.subcore_barrier()
        pltpu.sync_copy(acc_sh.at[pl.ds(sid * nz, nz)], row_v)
        pltpu.sync_copy(row_v, out_hbm.at[pl.ds(cid * NP + sid * nz, nz)])

    return deg_kernel


def _make_scatter_kernel(NP, T, D):
    nz = NP // NS
    q, r = divmod(T, NW)
    assert q % 2 == 0 and q >= 4

    @functools.partial(
        pl.kernel,
        out_type=jax.ShapeDtypeStruct((NC, NP, D), jnp.float32),
        mesh=_mesh(),
        scratch_types=[
            pltpu.VMEM((4, CHUNK), jnp.int32),
            pltpu.VMEM((CHUNK, D), jnp.float32),
            pltpu.VMEM((CHUNK, D), jnp.float32),
            pltpu.VMEM_SHARED((NP, D), jnp.float32),
        ] + [pltpu.SemaphoreType.DMA] * 4,
    )
    def scat_kernel(h_hbm, sdx_hbm, zeros_hbm, out_hbm,
                    idx_v, rows0, rows1, acc_sh,
                    g0, g1, s0, s1):
        cid = lax.axis_index("c")
        sid = lax.axis_index("s")
        wid = sid * NC + cid
        ld = tuple(idx_v.at[pl.ds(2 * b, 2)] for b in range(2))
        srci = tuple(idx_v.at[2 * b] for b in range(2))
        dsti = tuple(idx_v.at[2 * b + 1] for b in range(2))
        rows = (rows0, rows1)
        gsem = (g0, g1)
        ssem = (s0, s1)
        base = q * wid + jnp.minimum(wid, r)

        def load_gather(cc, b):
            pltpu.sync_copy(sdx_hbm.at[base + cc], ld[b])
            pltpu.async_copy(h_hbm.at[srci[b]], rows[b], gsem[b])

        def wait_gather(b):
            pltpu.make_async_copy(h_hbm.at[srci[b]], rows[b], gsem[b]).wait()

        def start_scatter(b):
            pltpu.async_copy(rows[b], acc_sh.at[dsti[b]], ssem[b], add=True)

        def wait_scatter(b):
            pltpu.make_async_copy(rows[b], acc_sh.at[dsti[b]], ssem[b]).wait()

        load_gather(0, 0)
        pltpu.sync_copy(zeros_hbm.at[pl.ds(sid * nz, nz)],
                        acc_sh.at[pl.ds(sid * nz, nz)])
        plsc.subcore_barrier()

        wait_gather(0)
        start_scatter(0)
        load_gather(1, 1)

        @pl.loop(0, (q - 2) // 2)
        def _(i):
            c = 1 + i * 2
            for db in range(2):
                cc = c + db
                b = (1 + db) % 2
                wait_gather(b)
                start_scatter(b)
                wait_scatter(1 - b)
                load_gather(cc + 1, 1 - b)

        wait_gather(1)
        start_scatter(1)
        wait_scatter(0)
        if r:
            @pl.when(wid < r)
            def _():
                pltpu.sync_copy(sdx_hbm.at[base + q], ld[0])
                pltpu.async_copy(h_hbm.at[srci[0]], rows[0], gsem[0]).wait()
                pltpu.sync_copy(rows[0], acc_sh.at[dsti[0]], add=True)
        wait_scatter(1)

        plsc.subcore_barrier()
        pltpu.sync_copy(acc_sh.at[pl.ds(sid * nz, nz)],
                        out_hbm.at[cid, pl.ds(sid * nz, nz)])

    return scat_kernel


def _matmul(x, W):
    NP, D = x.shape
    B = NP // 8

    def body(x_ref, w_ref, o_ref):
        o_ref[...] = jnp.dot(x_ref[...], w_ref[...],
                             preferred_element_type=jnp.float32)

    return pl.pallas_call(
        body,
        grid=(8,),
        in_specs=[pl.BlockSpec((B, D), lambda i: (i, 0)),
                  pl.BlockSpec((D, D), lambda i: (0, 0))],
        out_specs=pl.BlockSpec((B, D), lambda i: (i, 0)),
        out_shape=jax.ShapeDtypeStruct((NP, D), jnp.float32),
    )(x, W)


def _prep(degf, g1):
    _, NPL, _ = degf.shape
    NP, D = g1.shape
    B = NP // 10
    BL = NPL // 10

    def body(d_ref, g_ref, di_ref, h_ref):
        d = d_ref[0] + d_ref[1]
        di = lax.rsqrt(d + 1.0)
        dit = di.T
        dib = jnp.concatenate(
            [jnp.broadcast_to(dit[:, a:a + 1], (128, D)) for a in range(BL)],
            axis=0)
        di_ref[...] = dib
        h_ref[...] = dib * g_ref[...]

    return pl.pallas_call(
        body,
        grid=(10,),
        in_specs=[pl.BlockSpec((2, BL, 128), lambda i: (0, i, 0)),
                  pl.BlockSpec((B, D), lambda i: (i, 0))],
        out_specs=[pl.BlockSpec((B, D), lambda i: (i, 0)),
                   pl.BlockSpec((B, D), lambda i: (i, 0))],
        out_shape=[jax.ShapeDtypeStruct((NP, D), jnp.float32),
                   jax.ShapeDtypeStruct((NP, D), jnp.float32)],
    )(degf, g1)


def _mid(y, ht, dinvb, b, W):
    _, NP, D = y.shape
    B = NP // 8

    def body(y_ref, h_ref, di_ref, b_ref, w_ref, o_ref):
        s = y_ref[0] + y_ref[1] + h_ref[...]
        z = jnp.maximum(di_ref[...] * s + b_ref[...], 0.0)
        o_ref[...] = di_ref[...] * jnp.dot(z, w_ref[...],
                                           preferred_element_type=jnp.float32)

    return pl.pallas_call(
        body,
        grid=(8,),
        in_specs=[pl.BlockSpec((2, B, D), lambda i: (0, i, 0)),
                  pl.BlockSpec((B, D), lambda i: (i, 0)),
                  pl.BlockSpec((B, D), lambda i: (i, 0)),
                  pl.BlockSpec((1, D), lambda i: (0, 0)),
                  pl.BlockSpec((D, D), lambda i: (0, 0))],
        out_specs=pl.BlockSpec((B, D), lambda i: (i, 0)),
        out_shape=jax.ShapeDtypeStruct((NP, D), jnp.float32),
    )(y, ht, dinvb, b, W)


def _final(y, ht, dinvb, b, N):
    _, NP, D = y.shape
    B = N // 10

    def body(y_ref, h_ref, di_ref, b_ref, o_ref):
        s = y_ref[0] + y_ref[1] + h_ref[...]
        o_ref[...] = di_ref[...] * s + b_ref[...]

    return pl.pallas_call(
        body,
        grid=(10,),
        in_specs=[pl.BlockSpec((2, B, D), lambda i: (0, i, 0)),
                  pl.BlockSpec((B, D), lambda i: (i, 0)),
                  pl.BlockSpec((B, D), lambda i: (i, 0)),
                  pl.BlockSpec((1, D), lambda i: (0, 0))],
        out_specs=pl.BlockSpec((B, D), lambda i: (i, 0)),
        out_shape=jax.ShapeDtypeStruct((N, D), jnp.float32),
    )(y, ht, dinvb, b)


def kernel(x, edge_index, W1, b1, W2, b2):
    N, D = x.shape
    E = edge_index.shape[1]
    NP = -(-(N + PAD_ROWS) // 1024) * 1024
    T = E // CHUNK

    ei = edge_index.astype(jnp.int32)
    sdx = ei.reshape(2, T, CHUNK).transpose(1, 0, 2)

    xp = jnp.pad(x, ((0, NP - N), (0, 0)))
    zeros2 = jnp.zeros((NP, D), jnp.float32)

    deg_k = _make_deg_kernel(NP, T)
    scat_k = _make_scatter_kernel(NP, T, D)

    degf = deg_k(sdx).reshape(NC, NP // 128, 128)
    g1 = _matmul(xp, W1)
    dinvb, h1t = _prep(degf, g1)
    y1 = scat_k(h1t, sdx, zeros2)
    h2t = _mid(y1, h1t, dinvb, b1.reshape(1, D), W2)
    y2 = scat_k(h2t, sdx, zeros2)
    return _final(y2, h2t, dinvb, b2.reshape(1, D), N)


# speedup vs baseline: 1.2410x; 1.2410x over previous
import functools

import jax
import jax.numpy as jnp
from jax import lax
from jax.experimental import pallas as pl
from jax.experimental.pallas import tpu as pltpu
from jax.experimental.pallas import tpu_sc as plsc

NC = 2
NS = 16
NW = NC * NS
LANES = 16
CHUNK = 128
SCH = 64
RING = 4
PAD_ROWS = 16


def _mesh():
    return plsc.VectorSubcoreMesh(core_axis_name="c", subcore_axis_name="s")


def _make_deg_kernel(NP, T):
    nz = NP // NS
    q, r = divmod(T, NW)
    NCHR = (q // 4) * 4

    @functools.partial(
        pl.kernel,
        out_type=jax.ShapeDtypeStruct((NC * NP,), jnp.float32),
        mesh=_mesh(),
        scratch_types=[
            pltpu.VMEM((8, CHUNK), jnp.int32),
            pltpu.VMEM((CHUNK,), jnp.float32),
            pltpu.VMEM((NP // NS,), jnp.float32),
            pltpu.VMEM_SHARED((NP,), jnp.float32),
            pltpu.SemaphoreType.DMA,
            pltpu.SemaphoreType.DMA,
            pltpu.SemaphoreType.DMA,
            pltpu.SemaphoreType.DMA,
        ],
    )
    def deg_kernel(sdx_hbm, out_hbm, idx_v, ones_v, row_v, acc_sh,
                   s0, s1, s2, s3):
        cid = lax.axis_index("c")
        sid = lax.axis_index("s")
        wid = sid * NC + cid
        ld = tuple(idx_v.at[pl.ds(2 * b, 2)] for b in range(4))
        dsti = tuple(idx_v.at[2 * b + 1] for b in range(4))
        sems = (s0, s1, s2, s3)
        base = q * wid + jnp.minimum(wid, r)

        @pl.loop(0, CHUNK // LANES)
        def _(i):
            ones_v[pl.ds(i * LANES, LANES)] = jnp.full((LANES,), 1.0, jnp.float32)

        @pl.loop(0, nz // LANES)
        def _(i):
            row_v[pl.ds(i * LANES, LANES)] = jnp.zeros((LANES,), jnp.float32)

        pltpu.sync_copy(row_v, acc_sh.at[pl.ds(sid * nz, nz)])
        plsc.subcore_barrier()

        for b in range(4):
            pltpu.sync_copy(sdx_hbm.at[base + b], ld[b])
            pltpu.async_copy(ones_v, acc_sh.at[dsti[b]], sems[b], add=True)

        @pl.loop(0, (NCHR - 4) // 4)
        def _(i):
            c = i * 4
            for b in range(4):
                cc = c + b
                pltpu.make_async_copy(ones_v, acc_sh.at[dsti[b]],
                                      sems[b]).wait()
                pltpu.sync_copy(sdx_hbm.at[base + cc + 4], ld[b])
                pltpu.async_copy(ones_v, acc_sh.at[dsti[b]], sems[b],
                                 add=True)

        for b in range(4):
            pltpu.make_async_copy(ones_v, acc_sh.at[dsti[b]], sems[b]).wait()

        for cc in range(NCHR, q):
            pltpu.sync_copy(sdx_hbm.at[base + cc], ld[0])
            pltpu.sync_copy(ones_v, acc_sh.at[dsti[0]], add=True)
        if r:
            @pl.when(wid < r)
            def _():
                pltpu.sync_copy(sdx_hbm.at[base + q], ld[0])
                pltpu.sync_copy(ones_v, acc_sh.at[dsti[0]], add=True)

        plsc.subcore_barrier()
        pltpu.sync_copy(acc_sh.at[pl.ds(sid * nz, nz)], row_v)
        pltpu.sync_copy(row_v, out_hbm.at[pl.ds(cid * NP + sid * nz, nz)])

    return deg_kernel


def _make_scatter_kernel(NP, T, D):
    nz = NP // NS
    q, r = divmod(T, NW)
    assert q % 2 == 0 and q >= 4

    HC = CHUNK // 2
    HN = 2 * q

    @functools.partial(
        pl.kernel,
        out_type=jax.ShapeDtypeStruct((NC, NP, D), jnp.float32),
        mesh=_mesh(),
        scratch_types=[
            pltpu.VMEM((RING, 2, 2, HC), jnp.int32),
            pltpu.VMEM((HC, D), jnp.float32),
            pltpu.VMEM((HC, D), jnp.float32),
            pltpu.VMEM((HC, D), jnp.float32),
            pltpu.VMEM((HC, D), jnp.float32),
            pltpu.VMEM_SHARED((NP, D), jnp.float32),
        ] + [pltpu.SemaphoreType.DMA] * 8,
    )
    def scat_kernel(h_hbm, sdx_hbm, zeros_hbm, out_hbm,
                    idx_v, rows0, rows1, rows2, rows3, acc_sh,
                    g0, g1, g2, g3, s0, s1, s2, s3):
        cid = lax.axis_index("c")
        sid = lax.axis_index("s")
        wid = sid * NC + cid
        ld = tuple(idx_v.at[b] for b in range(RING))
        srci = tuple(idx_v.at[b, 0, b % 2] for b in range(RING))
        dsti = tuple(idx_v.at[b, 1, b % 2] for b in range(RING))
        rows = (rows0, rows1, rows2, rows3)
        gsem = (g0, g1, g2, g3)
        ssem = (s0, s1, s2, s3)
        base = q * wid + jnp.minimum(wid, r)

        def load_gather(row, b):
            pltpu.sync_copy(sdx_hbm.at[base + row], ld[b])
            pltpu.async_copy(h_hbm.at[srci[b]], rows[b], gsem[b])

        def wait_gather_scatter(b):
            pltpu.make_async_copy(h_hbm.at[srci[b]], rows[b], gsem[b]).wait()
            pltpu.async_copy(rows[b], acc_sh.at[dsti[b]], ssem[b], add=True)

        def wait_scatter(b):
            pltpu.make_async_copy(rows[b], acc_sh.at[dsti[b]], ssem[b]).wait()

        for b in range(2):
            load_gather(0, b)

        pltpu.sync_copy(zeros_hbm.at[pl.ds(sid * nz, nz)],
                        acc_sh.at[pl.ds(sid * nz, nz)])
        plsc.subcore_barrier()

        for cc in range(2):
            load_gather(1, cc + 2)
            wait_gather_scatter(cc)

        @pl.loop(0, (HN - 4) // 4)
        def _(i):
            c = 2 + i * 4
            row2 = c // 2 + 1
            for db in range(4):
                b = (2 + db) % 4
                bL = db
                wait_scatter(bL)
                load_gather(row2 + (db // 2), bL)
                wait_gather_scatter(b)

        for db in range(2):
            wait_gather_scatter(2 + db)
        for b in range(4):
            wait_scatter(b)

        if r:
            @pl.when(wid < r)
            def _():
                for b in range(2):
                    pltpu.sync_copy(sdx_hbm.at[base + q], ld[b])
                    pltpu.async_copy(h_hbm.at[srci[b]], rows[b],
                                     gsem[b]).wait()
                    pltpu.sync_copy(rows[b], acc_sh.at[dsti[b]], add=True)

        plsc.subcore_barrier()
        pltpu.sync_copy(acc_sh.at[pl.ds(sid * nz, nz)],
                        out_hbm.at[cid, pl.ds(sid * nz, nz)])

    return scat_kernel


def _matmul(x, W):
    NP, D = x.shape
    B = NP // 8

    def body(x_ref, w_ref, o_ref):
        o_ref[...] = jnp.dot(x_ref[...], w_ref[...],
                             preferred_element_type=jnp.float32)

    return pl.pallas_call(
        body,
        grid=(8,),
        in_specs=[pl.BlockSpec((B, D), lambda i: (i, 0)),
                  pl.BlockSpec((D, D), lambda i: (0, 0))],
        out_specs=pl.BlockSpec((B, D), lambda i: (i, 0)),
        out_shape=jax.ShapeDtypeStruct((NP, D), jnp.float32),
    )(x, W)


def _prep(degf, g1):
    _, NPL, _ = degf.shape
    NP, D = g1.shape
    B = NP // 10
    BL = NPL // 10

    def body(d_ref, g_ref, di_ref, h_ref):
        d = d_ref[0] + d_ref[1]
        di = lax.rsqrt(d + 1.0)
        dit = di.T
        dib = jnp.concatenate(
            [jnp.broadcast_to(dit[:, a:a + 1], (128, D)) for a in range(BL)],
            axis=0)
        di_ref[...] = dib
        h_ref[...] = dib * g_ref[...]

    return pl.pallas_call(
        body,
        grid=(10,),
        in_specs=[pl.BlockSpec((2, BL, 128), lambda i: (0, i, 0)),
                  pl.BlockSpec((B, D), lambda i: (i, 0))],
        out_specs=[pl.BlockSpec((B, D), lambda i: (i, 0)),
                   pl.BlockSpec((B, D), lambda i: (i, 0))],
        out_shape=[jax.ShapeDtypeStruct((NP, D), jnp.float32),
                   jax.ShapeDtypeStruct((NP, D), jnp.float32)],
    )(degf, g1)


def _mid(y, ht, dinvb, b, W):
    _, NP, D = y.shape
    B = NP // 8

    def body(y_ref, h_ref, di_ref, b_ref, w_ref, o_ref):
        s = y_ref[0] + y_ref[1] + h_ref[...]
        z = jnp.maximum(di_ref[...] * s + b_ref[...], 0.0)
        o_ref[...] = di_ref[...] * jnp.dot(z, w_ref[...],
                                           preferred_element_type=jnp.float32)

    return pl.pallas_call(
        body,
        grid=(8,),
        in_specs=[pl.BlockSpec((2, B, D), lambda i: (0, i, 0)),
                  pl.BlockSpec((B, D), lambda i: (i, 0)),
                  pl.BlockSpec((B, D), lambda i: (i, 0)),
                  pl.BlockSpec((1, D), lambda i: (0, 0)),
                  pl.BlockSpec((D, D), lambda i: (0, 0))],
        out_specs=pl.BlockSpec((B, D), lambda i: (i, 0)),
        out_shape=jax.ShapeDtypeStruct((NP, D), jnp.float32),
    )(y, ht, dinvb, b, W)


def _final(y, ht, dinvb, b, N):
    _, NP, D = y.shape
    B = N // 10

    def body(y_ref, h_ref, di_ref, b_ref, o_ref):
        s = y_ref[0] + y_ref[1] + h_ref[...]
        o_ref[...] = di_ref[...] * s + b_ref[...]

    return pl.pallas_call(
        body,
        grid=(10,),
        in_specs=[pl.BlockSpec((2, B, D), lambda i: (0, i, 0)),
                  pl.BlockSpec((B, D), lambda i: (i, 0)),
                  pl.BlockSpec((B, D), lambda i: (i, 0)),
                  pl.BlockSpec((1, D), lambda i: (0, 0))],
        out_specs=pl.BlockSpec((B, D), lambda i: (i, 0)),
        out_shape=jax.ShapeDtypeStruct((N, D), jnp.float32),
    )(y, ht, dinvb, b)


def kernel(x, edge_index, W1, b1, W2, b2):
    N, D = x.shape
    E = edge_index.shape[1]
    NP = -(-(N + PAD_ROWS) // 1024) * 1024
    T = E // CHUNK

    ei = edge_index.astype(jnp.int32)
    sdx = ei.reshape(2, T, CHUNK).transpose(1, 0, 2)
    sdx4 = sdx.reshape(T, 2, 2, CHUNK // 2)

    xp = jnp.pad(x, ((0, NP - N), (0, 0)))
    zeros2 = jnp.zeros((NP, D), jnp.float32)

    deg_k = _make_deg_kernel(NP, T)
    scat_k = _make_scatter_kernel(NP, T, D)

    degf = deg_k(sdx).reshape(NC, NP // 128, 128)
    g1 = _matmul(xp, W1)
    dinvb, h1t = _prep(degf, g1)
    y1 = scat_k(h1t, sdx4, zeros2)
    h2t = _mid(y1, h1t, dinvb, b1.reshape(1, D), W2)
    y2 = scat_k(h2t, sdx4, zeros2)
    return _final(y2, h2t, dinvb, b2.reshape(1, D), N)


# speedup vs baseline: 1.2888x; 1.0385x over previous
import functools

import jax
import jax.numpy as jnp
from jax import lax
from jax.experimental import pallas as pl
from jax.experimental.pallas import tpu as pltpu
from jax.experimental.pallas import tpu_sc as plsc

NC = 2
NS = 16
NW = NC * NS
LANES = 16
CHUNK = 128
SCH = 64
RING = 4
PAD_ROWS = 16


def _mesh():
    return plsc.VectorSubcoreMesh(core_axis_name="c", subcore_axis_name="s")


def _make_deg_kernel(NP, T):
    nz = NP // NS
    q, r = divmod(T, NW)
    NCHR = (q // 4) * 4

    @functools.partial(
        pl.kernel,
        out_type=jax.ShapeDtypeStruct((NC * NP,), jnp.float32),
        mesh=_mesh(),
        scratch_types=[
            pltpu.VMEM((8, CHUNK), jnp.int32),
            pltpu.VMEM((CHUNK,), jnp.float32),
            pltpu.VMEM((NP // NS,), jnp.float32),
            pltpu.VMEM_SHARED((NP,), jnp.float32),
            pltpu.SemaphoreType.DMA,
            pltpu.SemaphoreType.DMA,
            pltpu.SemaphoreType.DMA,
            pltpu.SemaphoreType.DMA,
        ],
    )
    def deg_kernel(sdx_hbm, out_hbm, idx_v, ones_v, row_v, acc_sh,
                   s0, s1, s2, s3):
        cid = lax.axis_index("c")
        sid = lax.axis_index("s")
        wid = sid * NC + cid
        ld = tuple(idx_v.at[pl.ds(2 * b, 2)] for b in range(4))
        dsti = tuple(idx_v.at[2 * b + 1] for b in range(4))
        sems = (s0, s1, s2, s3)
        base = q * wid + jnp.minimum(wid, r)

        @pl.loop(0, CHUNK // LANES)
        def _(i):
            ones_v[pl.ds(i * LANES, LANES)] = jnp.full((LANES,), 1.0, jnp.float32)

        @pl.loop(0, nz // LANES)
        def _(i):
            row_v[pl.ds(i * LANES, LANES)] = jnp.zeros((LANES,), jnp.float32)

        pltpu.sync_copy(row_v, acc_sh.at[pl.ds(sid * nz, nz)])
        plsc.subcore_barrier()

        for b in range(4):
            pltpu.sync_copy(sdx_hbm.at[base + b], ld[b])
            pltpu.async_copy(ones_v, acc_sh.at[dsti[b]], sems[b], add=True)

        @pl.loop(0, (NCHR - 4) // 4)
        def _(i):
            c = i * 4
            for b in range(4):
                cc = c + b
                pltpu.make_async_copy(ones_v, acc_sh.at[dsti[b]],
                                      sems[b]).wait()
                pltpu.sync_copy(sdx_hbm.at[base + cc + 4], ld[b])
                pltpu.async_copy(ones_v, acc_sh.at[dsti[b]], sems[b],
                                 add=True)

        for b in range(4):
            pltpu.make_async_copy(ones_v, acc_sh.at[dsti[b]], sems[b]).wait()

        for cc in range(NCHR, q):
            pltpu.sync_copy(sdx_hbm.at[base + cc], ld[0])
            pltpu.sync_copy(ones_v, acc_sh.at[dsti[0]], add=True)
        if r:
            @pl.when(wid < r)
            def _():
                pltpu.sync_copy(sdx_hbm.at[base + q], ld[0])
                pltpu.sync_copy(ones_v, acc_sh.at[dsti[0]], add=True)

        plsc.subcore_barrier()
        pltpu.sync_copy(acc_sh.at[pl.ds(sid * nz, nz)], row_v)
        pltpu.sync_copy(row_v, out_hbm.at[pl.ds(cid * NP + sid * nz, nz)])

    return deg_kernel


def _make_scatter_kernel(NP, T, D):
    nz = NP // NS
    q, r = divmod(T, NW)
    assert q % 2 == 0 and q >= 4

    NCH = 2 * q

    @functools.partial(
        pl.kernel,
        out_type=jax.ShapeDtypeStruct((NC, NP, D), jnp.float32),
        mesh=_mesh(),
        scratch_types=[
            pltpu.VMEM((2 * RING, SCH), jnp.int32),
            pltpu.VMEM((SCH, D), jnp.float32),
            pltpu.VMEM((SCH, D), jnp.float32),
            pltpu.VMEM((SCH, D), jnp.float32),
            pltpu.VMEM((SCH, D), jnp.float32),
            pltpu.VMEM_SHARED((NP, D), jnp.float32),
        ] + [pltpu.SemaphoreType.DMA] * 8,
    )
    def scat_kernel(h_hbm, sdx_hbm, zeros_hbm, out_hbm,
                    idx_v, rows0, rows1, rows2, rows3, acc_sh,
                    g0, g1, g2, g3, s0, s1, s2, s3):
        cid = lax.axis_index("c")
        sid = lax.axis_index("s")
        wid = sid * NC + cid
        ld = tuple(idx_v.at[pl.ds(2 * b, 2)] for b in range(4))
        srci = tuple(idx_v.at[2 * b] for b in range(4))
        dsti = tuple(idx_v.at[2 * b + 1] for b in range(4))
        rows = (rows0, rows1, rows2, rows3)
        gsem = (g0, g1, g2, g3)
        ssem = (s0, s1, s2, s3)
        base = NCH * wid + 2 * jnp.minimum(wid, r)

        def load_gather(cc, b):
            pltpu.sync_copy(sdx_hbm.at[base + cc], ld[b])
            pltpu.async_copy(h_hbm.at[srci[b]], rows[b], gsem[b])

        def wait_gather_scatter(cc, b):
            pltpu.make_async_copy(h_hbm.at[srci[b]], rows[b], gsem[b]).wait()
            pltpu.async_copy(rows[b], acc_sh.at[dsti[b]], ssem[b], add=True)

        def wait_scatter(b):
            pltpu.make_async_copy(rows[b], acc_sh.at[dsti[b]], ssem[b]).wait()

        for b in range(2):
            load_gather(b, b)

        pltpu.sync_copy(zeros_hbm.at[pl.ds(sid * nz, nz)],
                        acc_sh.at[pl.ds(sid * nz, nz)])
        plsc.subcore_barrier()

        for cc in range(2):
            load_gather(cc + 2, cc + 2)
            wait_gather_scatter(cc, cc)

        @pl.loop(0, (NCH - 4) // 4)
        def _(i):
            c = 2 + i * 4
            for db in range(4):
                cc = c + db
                b = (2 + db) % 4
                bL = db
                wait_scatter(bL)
                load_gather(cc + 2, bL)
                wait_gather_scatter(cc, b)

        for db in range(2):
            wait_gather_scatter(NCH - 2 + db, 2 + db)
        for b in range(4):
            wait_scatter(b)

        if r:
            @pl.when(wid < r)
            def _():
                for b in range(2):
                    pltpu.sync_copy(sdx_hbm.at[base + NCH + b], ld[b])
                    pltpu.async_copy(h_hbm.at[srci[b]], rows[b],
                                     gsem[b]).wait()
                    pltpu.sync_copy(rows[b], acc_sh.at[dsti[b]], add=True)

        plsc.subcore_barrier()
        pltpu.sync_copy(acc_sh.at[pl.ds(sid * nz, nz)],
                        out_hbm.at[cid, pl.ds(sid * nz, nz)])

    return scat_kernel


def _matmul(x, W):
    NP, D = x.shape
    B = NP // 8

    def body(x_ref, w_ref, o_ref):
        o_ref[...] = jnp.dot(x_ref[...], w_ref[...],
                             preferred_element_type=jnp.float32)

    return pl.pallas_call(
        body,
        grid=(8,),
        in_specs=[pl.BlockSpec((B, D), lambda i: (i, 0)),
                  pl.BlockSpec((D, D), lambda i: (0, 0))],
        out_specs=pl.BlockSpec((B, D), lambda i: (i, 0)),
        out_shape=jax.ShapeDtypeStruct((NP, D), jnp.float32),
    )(x, W)


def _prep(degf, g1):
    _, NPL, _ = degf.shape
    NP, D = g1.shape
    B = NP // 10
    BL = NPL // 10

    def body(d_ref, g_ref, di_ref, h_ref):
        d = d_ref[0] + d_ref[1]
        di = lax.rsqrt(d + 1.0)
        dit = di.T
        dib = jnp.concatenate(
            [jnp.broadcast_to(dit[:, a:a + 1], (128, D)) for a in range(BL)],
            axis=0)
        di_ref[...] = dib
        h_ref[...] = dib * g_ref[...]

    return pl.pallas_call(
        body,
        grid=(10,),
        in_specs=[pl.BlockSpec((2, BL, 128), lambda i: (0, i, 0)),
                  pl.BlockSpec((B, D), lambda i: (i, 0))],
        out_specs=[pl.BlockSpec((B, D), lambda i: (i, 0)),
                   pl.BlockSpec((B, D), lambda i: (i, 0))],
        out_shape=[jax.ShapeDtypeStruct((NP, D), jnp.float32),
                   jax.ShapeDtypeStruct((NP, D), jnp.float32)],
    )(degf, g1)


def _mid(y, ht, dinvb, b, W):
    _, NP, D = y.shape
    B = NP // 8

    def body(y_ref, h_ref, di_ref, b_ref, w_ref, o_ref):
        s = y_ref[0] + y_ref[1] + h_ref[...]
        z = jnp.maximum(di_ref[...] * s + b_ref[...], 0.0)
        o_ref[...] = di_ref[...] * jnp.dot(z, w_ref[...],
                                           preferred_element_type=jnp.float32)

    return pl.pallas_call(
        body,
        grid=(8,),
        in_specs=[pl.BlockSpec((2, B, D), lambda i: (0, i, 0)),
                  pl.BlockSpec((B, D), lambda i: (i, 0)),
                  pl.BlockSpec((B, D), lambda i: (i, 0)),
                  pl.BlockSpec((1, D), lambda i: (0, 0)),
                  pl.BlockSpec((D, D), lambda i: (0, 0))],
        out_specs=pl.BlockSpec((B, D), lambda i: (i, 0)),
        out_shape=jax.ShapeDtypeStruct((NP, D), jnp.float32),
    )(y, ht, dinvb, b, W)


def _final(y, ht, dinvb, b, N):
    _, NP, D = y.shape
    B = N // 10

    def body(y_ref, h_ref, di_ref, b_ref, o_ref):
        s = y_ref[0] + y_ref[1] + h_ref[...]
        o_ref[...] = di_ref[...] * s + b_ref[...]

    return pl.pallas_call(
        body,
        grid=(10,),
        in_specs=[pl.BlockSpec((2, B, D), lambda i: (0, i, 0)),
                  pl.BlockSpec((B, D), lambda i: (i, 0)),
                  pl.BlockSpec((B, D), lambda i: (i, 0)),
                  pl.BlockSpec((1, D), lambda i: (0, 0))],
        out_specs=pl.BlockSpec((B, D), lambda i: (i, 0)),
        out_shape=jax.ShapeDtypeStruct((N, D), jnp.float32),
    )(y, ht, dinvb, b)


def kernel(x, edge_index, W1, b1, W2, b2):
    N, D = x.shape
    E = edge_index.shape[1]
    NP = -(-(N + PAD_ROWS) // 1024) * 1024
    T = E // CHUNK

    ei = edge_index.astype(jnp.int32)
    sdx = ei.reshape(2, T, CHUNK).transpose(1, 0, 2)
    sdx64 = jnp.stack([ei[0].reshape(-1, SCH), ei[1].reshape(-1, SCH)], axis=1)

    xp = jnp.pad(x, ((0, NP - N), (0, 0)))
    zeros2 = jnp.zeros((NP, D), jnp.float32)

    deg_k = _make_deg_kernel(NP, T)
    scat_k = _make_scatter_kernel(NP, T, D)

    degf = deg_k(sdx).reshape(NC, NP // 128, 128)
    g1 = _matmul(xp, W1)
    dinvb, h1t = _prep(degf, g1)
    y1 = scat_k(h1t, sdx64, zeros2)
    h2t = _mid(y1, h1t, dinvb, b1.reshape(1, D), W2)
    y2 = scat_k(h2t, sdx64, zeros2)
    return _final(y2, h2t, dinvb, b2.reshape(1, D), N)


# speedup vs baseline: 1.3298x; 1.0319x over previous
import dataclasses
import functools

import jax
import jax.numpy as jnp
from jax import lax
from jax.experimental import pallas as pl
from jax.experimental.pallas import tpu as pltpu
from jax.experimental.pallas import tpu_sc as plsc

NC = 2
NS = 16
NW = NC * NS
LANES = 16
CHUNK = 128
SCH = 64
RING = 4
PAD_ROWS = 16


def _mesh():
    return plsc.VectorSubcoreMesh(core_axis_name="c", subcore_axis_name="s")


def _sc_compiler_params():
    cp = pltpu.CompilerParams()
    if "needs_layout_passes" in pltpu.CompilerParams.__dataclass_fields__:
        cp = dataclasses.replace(cp, needs_layout_passes=False)
    return cp


def _make_deg_kernel(NP, T):
    q, r = divmod(T, NW)
    NPH = NP // 2
    G = 13
    ngroups, grem = divmod(q, G)

    @functools.partial(
        pl.kernel,
        out_type=jax.ShapeDtypeStruct((NW * NP,), jnp.float32),
        mesh=_mesh(),
        scratch_types=[
            pltpu.VMEM((G, 2, CHUNK), jnp.int32),
            pltpu.VMEM((NPH,), jnp.float32),
        ],
        compiler_params=_sc_compiler_params(),
    )
    def deg_kernel(sdx_hbm, out_hbm, stg_v, hist_v):
        cid = lax.axis_index("c")
        sid = lax.axis_index("s")
        wid = sid * NC + cid
        base = q * wid + jnp.minimum(wid, r)
        ones = jnp.full((LANES,), 1.0, jnp.float32)

        def count_row(j, lo):
            for k in range(CHUNK // LANES):
                idx = stg_v[j, 1, pl.ds(k * LANES, LANES)]
                sh = idx - lo
                m = (sh >= 0) & (sh < NPH)
                plsc.addupdate_scatter(hist_v, [sh], ones, mask=m)

        for p in range(2):
            lo = p * NPH

            @pl.loop(0, NPH // LANES)
            def _(i):
                hist_v[pl.ds(i * LANES, LANES)] = jnp.zeros((LANES,),
                                                            jnp.float32)

            for g in range(ngroups):
                pltpu.sync_copy(sdx_hbm.at[pl.ds(base + g * G, G)], stg_v)

                @pl.loop(0, G)
                def _(j):
                    count_row(j, lo)

            if grem:
                pltpu.sync_copy(sdx_hbm.at[pl.ds(base + ngroups * G, grem)],
                                stg_v.at[pl.ds(0, grem)])

                @pl.loop(0, grem)
                def _(j):
                    count_row(j, lo)

            if r:
                @pl.when(wid < r)
                def _():
                    pltpu.sync_copy(sdx_hbm.at[pl.ds(base + q, 1)],
                                    stg_v.at[pl.ds(0, 1)])
                    count_row(0, lo)

            pltpu.sync_copy(hist_v, out_hbm.at[pl.ds(wid * NP + lo, NPH)])

    return deg_kernel


def _make_scatter_kernel(NP, T, D):
    nz = NP // NS
    q, r = divmod(T, NW)
    assert q % 2 == 0 and q >= 4

    NCH = 2 * q

    @functools.partial(
        pl.kernel,
        out_type=jax.ShapeDtypeStruct((NC, NP, D), jnp.float32),
        mesh=_mesh(),
        scratch_types=[
            pltpu.VMEM((2 * RING, SCH), jnp.int32),
            pltpu.VMEM((SCH, D), jnp.float32),
            pltpu.VMEM((SCH, D), jnp.float32),
            pltpu.VMEM((SCH, D), jnp.float32),
            pltpu.VMEM((SCH, D), jnp.float32),
            pltpu.VMEM_SHARED((NP, D), jnp.float32),
        ] + [pltpu.SemaphoreType.DMA] * 8,
    )
    def scat_kernel(h_hbm, sdx_hbm, zeros_hbm, out_hbm,
                    idx_v, rows0, rows1, rows2, rows3, acc_sh,
                    g0, g1, g2, g3, s0, s1, s2, s3):
        cid = lax.axis_index("c")
        sid = lax.axis_index("s")
        wid = sid * NC + cid
        ld = tuple(idx_v.at[pl.ds(2 * b, 2)] for b in range(4))
        srci = tuple(idx_v.at[2 * b] for b in range(4))
        dsti = tuple(idx_v.at[2 * b + 1] for b in range(4))
        rows = (rows0, rows1, rows2, rows3)
        gsem = (g0, g1, g2, g3)
        ssem = (s0, s1, s2, s3)
        base = NCH * wid + 2 * jnp.minimum(wid, r)

        def load_gather(cc, b):
            pltpu.sync_copy(sdx_hbm.at[base + cc], ld[b])
            pltpu.async_copy(h_hbm.at[srci[b]], rows[b], gsem[b])

        def wait_gather_scatter(cc, b):
            pltpu.make_async_copy(h_hbm.at[srci[b]], rows[b], gsem[b]).wait()
            pltpu.async_copy(rows[b], acc_sh.at[dsti[b]], ssem[b], add=True)

        def wait_scatter(b):
            pltpu.make_async_copy(rows[b], acc_sh.at[dsti[b]], ssem[b]).wait()

        for b in range(2):
            load_gather(b, b)

        pltpu.sync_copy(zeros_hbm.at[pl.ds(sid * nz, nz)],
                        acc_sh.at[pl.ds(sid * nz, nz)])
        plsc.subcore_barrier()

        for cc in range(2):
            load_gather(cc + 2, cc + 2)
            wait_gather_scatter(cc, cc)

        @pl.loop(0, (NCH - 4) // 4)
        def _(i):
            c = 2 + i * 4
            for db in range(4):
                cc = c + db
                b = (2 + db) % 4
                bL = db
                wait_scatter(bL)
                load_gather(cc + 2, bL)
                wait_gather_scatter(cc, b)

        for db in range(2):
            wait_gather_scatter(NCH - 2 + db, 2 + db)
        for b in range(4):
            wait_scatter(b)

        if r:
            @pl.when(wid < r)
            def _():
                for b in range(2):
                    pltpu.sync_copy(sdx_hbm.at[base + NCH + b], ld[b])
                    pltpu.async_copy(h_hbm.at[srci[b]], rows[b],
                                     gsem[b]).wait()
                    pltpu.sync_copy(rows[b], acc_sh.at[dsti[b]], add=True)

        plsc.subcore_barrier()
        pltpu.sync_copy(acc_sh.at[pl.ds(sid * nz, nz)],
                        out_hbm.at[cid, pl.ds(sid * nz, nz)])

    return scat_kernel


def _matmul(x, W):
    NP, D = x.shape
    B = NP // 8

    def body(x_ref, w_ref, o_ref):
        o_ref[...] = jnp.dot(x_ref[...], w_ref[...],
                             preferred_element_type=jnp.float32)

    return pl.pallas_call(
        body,
        grid=(8,),
        in_specs=[pl.BlockSpec((B, D), lambda i: (i, 0)),
                  pl.BlockSpec((D, D), lambda i: (0, 0))],
        out_specs=pl.BlockSpec((B, D), lambda i: (i, 0)),
        out_shape=jax.ShapeDtypeStruct((NP, D), jnp.float32),
    )(x, W)


def _prep(degf, g1):
    NTILES, NPL, _ = degf.shape
    NP, D = g1.shape
    B = NP // 10
    BL = NPL // 10

    def body(d_ref, g_ref, di_ref, h_ref):
        d = jnp.sum(d_ref[...], axis=0)
        di = lax.rsqrt(d + 1.0)
        dit = di.T
        dib = jnp.concatenate(
            [jnp.broadcast_to(dit[:, a:a + 1], (128, D)) for a in range(BL)],
            axis=0)
        di_ref[...] = dib
        h_ref[...] = dib * g_ref[...]

    return pl.pallas_call(
        body,
        grid=(10,),
        in_specs=[pl.BlockSpec((NTILES, BL, 128), lambda i: (0, i, 0)),
                  pl.BlockSpec((B, D), lambda i: (i, 0))],
        out_specs=[pl.BlockSpec((B, D), lambda i: (i, 0)),
                   pl.BlockSpec((B, D), lambda i: (i, 0))],
        out_shape=[jax.ShapeDtypeStruct((NP, D), jnp.float32),
                   jax.ShapeDtypeStruct((NP, D), jnp.float32)],
    )(degf, g1)


def _mid(y, ht, dinvb, b, W):
    _, NP, D = y.shape
    B = NP // 8

    def body(y_ref, h_ref, di_ref, b_ref, w_ref, o_ref):
        s = y_ref[0] + y_ref[1] + h_ref[...]
        z = jnp.maximum(di_ref[...] * s + b_ref[...], 0.0)
        o_ref[...] = di_ref[...] * jnp.dot(z, w_ref[...],
                                           preferred_element_type=jnp.float32)

    return pl.pallas_call(
        body,
        grid=(8,),
        in_specs=[pl.BlockSpec((2, B, D), lambda i: (0, i, 0)),
                  pl.BlockSpec((B, D), lambda i: (i, 0)),
                  pl.BlockSpec((B, D), lambda i: (i, 0)),
                  pl.BlockSpec((1, D), lambda i: (0, 0)),
                  pl.BlockSpec((D, D), lambda i: (0, 0))],
        out_specs=pl.BlockSpec((B, D), lambda i: (i, 0)),
        out_shape=jax.ShapeDtypeStruct((NP, D), jnp.float32),
    )(y, ht, dinvb, b, W)


def _final(y, ht, dinvb, b, N):
    _, NP, D = y.shape
    B = N // 10

    def body(y_ref, h_ref, di_ref, b_ref, o_ref):
        s = y_ref[0] + y_ref[1] + h_ref[...]
        o_ref[...] = di_ref[...] * s + b_ref[...]

    return pl.pallas_call(
        body,
        grid=(10,),
        in_specs=[pl.BlockSpec((2, B, D), lambda i: (0, i, 0)),
                  pl.BlockSpec((B, D), lambda i: (i, 0)),
                  pl.BlockSpec((B, D), lambda i: (i, 0)),
                  pl.BlockSpec((1, D), lambda i: (0, 0))],
        out_specs=pl.BlockSpec((B, D), lambda i: (i, 0)),
        out_shape=jax.ShapeDtypeStruct((N, D), jnp.float32),
    )(y, ht, dinvb, b)


def kernel(x, edge_index, W1, b1, W2, b2):
    N, D = x.shape
    E = edge_index.shape[1]
    NP = -(-(N + PAD_ROWS) // 1024) * 1024
    T = E // CHUNK

    ei = edge_index.astype(jnp.int32)
    sdx = ei.reshape(2, T, CHUNK).transpose(1, 0, 2)
    sdx64 = jnp.stack([ei[0].reshape(-1, SCH), ei[1].reshape(-1, SCH)], axis=1)

    xp = jnp.pad(x, ((0, NP - N), (0, 0)))
    zeros2 = jnp.zeros((NP, D), jnp.float32)

    deg_k = _make_deg_kernel(NP, T)
    scat_k = _make_scatter_kernel(NP, T, D)

    degf = deg_k(sdx).reshape(NW, NP // 128, 128)
    g1 = _matmul(xp, W1)
    dinvb, h1t = _prep(degf, g1)
    y1 = scat_k(h1t, sdx64, zeros2)
    h2t = _mid(y1, h1t, dinvb, b1.reshape(1, D), W2)
    y2 = scat_k(h2t, sdx64, zeros2)
    return _final(y2, h2t, dinvb, b2.reshape(1, D), N)


# speedup vs baseline: 1.3623x; 1.0244x over previous
import dataclasses
import functools

import jax
import jax.numpy as jnp
from jax import lax
from jax.experimental import pallas as pl
from jax.experimental.pallas import tpu as pltpu
from jax.experimental.pallas import tpu_sc as plsc

NC = 2
NS = 16
NW = NC * NS
LANES = 16
CHUNK = 128
SCH = 64
RING = 4
PAD_ROWS = 16


def _mesh():
    return plsc.VectorSubcoreMesh(core_axis_name="c", subcore_axis_name="s")


def _sc_compiler_params():
    cp = pltpu.CompilerParams()
    if "needs_layout_passes" in pltpu.CompilerParams.__dataclass_fields__:
        cp = dataclasses.replace(cp, needs_layout_passes=False)
    return cp


def _make_deg_kernel(NP, T):
    q, r = divmod(T, NW)
    NPH = NP // 2
    G = 13
    ngroups, grem = divmod(q, G)

    @functools.partial(
        pl.kernel,
        out_type=jax.ShapeDtypeStruct((NW * NP,), jnp.float32),
        mesh=_mesh(),
        scratch_types=[
            pltpu.VMEM((G, 2, CHUNK), jnp.int32),
            pltpu.VMEM((NPH,), jnp.float32),
        ],
        compiler_params=_sc_compiler_params(),
    )
    def deg_kernel(sdx_hbm, out_hbm, stg_v, hist_v):
        cid = lax.axis_index("c")
        sid = lax.axis_index("s")
        wid = sid * NC + cid
        base = q * wid + jnp.minimum(wid, r)
        ones = jnp.full((LANES,), 1.0, jnp.float32)

        def count_row(j, lo):
            for k in range(CHUNK // LANES):
                idx = stg_v[j, 1, pl.ds(k * LANES, LANES)]
                sh = idx - lo
                m = (sh >= 0) & (sh < NPH)
                plsc.addupdate_scatter(hist_v, [sh], ones, mask=m)

        for p in range(2):
            lo = p * NPH

            @pl.loop(0, NPH // LANES)
            def _(i):
                hist_v[pl.ds(i * LANES, LANES)] = jnp.zeros((LANES,),
                                                            jnp.float32)

            for g in range(ngroups):
                pltpu.sync_copy(sdx_hbm.at[pl.ds(base + g * G, G)], stg_v)

                @pl.loop(0, G)
                def _(j):
                    count_row(j, lo)

            if grem:
                pltpu.sync_copy(sdx_hbm.at[pl.ds(base + ngroups * G, grem)],
                                stg_v.at[pl.ds(0, grem)])

                @pl.loop(0, grem)
                def _(j):
                    count_row(j, lo)

            if r:
                @pl.when(wid < r)
                def _():
                    pltpu.sync_copy(sdx_hbm.at[pl.ds(base + q, 1)],
                                    stg_v.at[pl.ds(0, 1)])
                    count_row(0, lo)

            pltpu.sync_copy(hist_v, out_hbm.at[pl.ds(wid * NP + lo, NPH)])

    return deg_kernel


def _make_scatter_kernel(NP, T, D):
    nz = NP // NS
    q, r = divmod(T, NW)
    assert q % 2 == 0 and q >= 4

    NCH = 2 * q

    @functools.partial(
        pl.kernel,
        out_type=jax.ShapeDtypeStruct((NC, NP, D), jnp.float32),
        mesh=_mesh(),
        scratch_types=[
            pltpu.VMEM((2 * RING, SCH), jnp.int32),
            pltpu.VMEM((SCH, D), jnp.float32),
            pltpu.VMEM((SCH, D), jnp.float32),
            pltpu.VMEM((SCH, D), jnp.float32),
            pltpu.VMEM((SCH, D), jnp.float32),
            pltpu.VMEM_SHARED((NP, D), jnp.float32),
        ] + [pltpu.SemaphoreType.DMA] * 8,
    )
    def scat_kernel(h_hbm, sdx_hbm, zeros_hbm, out_hbm,
                    idx_v, rows0, rows1, rows2, rows3, acc_sh,
                    g0, g1, g2, g3, s0, s1, s2, s3):
        cid = lax.axis_index("c")
        sid = lax.axis_index("s")
        wid = sid * NC + cid
        ld = tuple(idx_v.at[pl.ds(2 * b, 2)] for b in range(4))
        srci = tuple(idx_v.at[2 * b] for b in range(4))
        dsti = tuple(idx_v.at[2 * b + 1] for b in range(4))
        rows = (rows0, rows1, rows2, rows3)
        gsem = (g0, g1, g2, g3)
        ssem = (s0, s1, s2, s3)
        base = NCH * wid + 2 * jnp.minimum(wid, r)

        def load_gather(cc, b):
            pltpu.sync_copy(sdx_hbm.at[base + cc], ld[b])
            pltpu.async_copy(h_hbm.at[srci[b]], rows[b], gsem[b])

        def wait_gather_scatter(cc, b):
            pltpu.make_async_copy(h_hbm.at[srci[b]], rows[b], gsem[b]).wait()
            pltpu.async_copy(rows[b], acc_sh.at[dsti[b]], ssem[b], add=True)

        def wait_scatter(b):
            pltpu.make_async_copy(rows[b], acc_sh.at[dsti[b]], ssem[b]).wait()

        for b in range(2):
            load_gather(b, b)

        pltpu.sync_copy(zeros_hbm.at[pl.ds(sid * nz, nz)],
                        acc_sh.at[pl.ds(sid * nz, nz)])
        plsc.subcore_barrier()

        for cc in range(2):
            load_gather(cc + 2, cc + 2)
            wait_gather_scatter(cc, cc)

        @pl.loop(0, (NCH - 4) // 4)
        def _(i):
            c = 2 + i * 4
            for db in range(4):
                cc = c + db
                b = (2 + db) % 4
                bL = db
                wait_scatter(bL)
                load_gather(cc + 2, bL)
                wait_gather_scatter(cc, b)

        for db in range(2):
            wait_gather_scatter(NCH - 2 + db, 2 + db)
        for b in range(4):
            wait_scatter(b)

        if r:
            @pl.when(wid < r)
            def _():
                for b in range(2):
                    pltpu.sync_copy(sdx_hbm.at[base + NCH + b], ld[b])
                    pltpu.async_copy(h_hbm.at[srci[b]], rows[b],
                                     gsem[b]).wait()
                    pltpu.sync_copy(rows[b], acc_sh.at[dsti[b]], add=True)

        plsc.subcore_barrier()
        pltpu.sync_copy(acc_sh.at[pl.ds(sid * nz, nz)],
                        out_hbm.at[cid, pl.ds(sid * nz, nz)])

    return scat_kernel


def _matmul(x, W):
    NP, D = x.shape
    B = NP // 8

    def body(x_ref, w_ref, o_ref):
        o_ref[...] = jnp.dot(x_ref[...], w_ref[...],
                             preferred_element_type=jnp.float32)

    return pl.pallas_call(
        body,
        grid=(8,),
        in_specs=[pl.BlockSpec((B, D), lambda i: (i, 0)),
                  pl.BlockSpec((D, D), lambda i: (0, 0))],
        out_specs=pl.BlockSpec((B, D), lambda i: (i, 0)),
        out_shape=jax.ShapeDtypeStruct((NP, D), jnp.float32),
    )(x, W)


def _prep(degf, g1):
    NTILES, NPL, _ = degf.shape
    NP, D = g1.shape
    B = NP // 10
    BL = NPL // 10

    def body(d_ref, g_ref, di_ref, h_ref):
        d = jnp.sum(d_ref[...], axis=0)
        di = lax.rsqrt(d + 1.0)
        dit = di.T
        dib = jnp.concatenate(
            [jnp.broadcast_to(dit[:, a:a + 1], (128, D)) for a in range(BL)],
            axis=0)
        di_ref[...] = dib
        h_ref[...] = dib * g_ref[...]

    return pl.pallas_call(
        body,
        grid=(10,),
        in_specs=[pl.BlockSpec((NTILES, BL, 128), lambda i: (0, i, 0)),
                  pl.BlockSpec((B, D), lambda i: (i, 0))],
        out_specs=[pl.BlockSpec((B, D), lambda i: (i, 0)),
                   pl.BlockSpec((B, D), lambda i: (i, 0))],
        out_shape=[jax.ShapeDtypeStruct((NP, D), jnp.float32),
                   jax.ShapeDtypeStruct((NP, D), jnp.float32)],
    )(degf, g1)


def _mid(y, ht, dinvb, b, W):
    _, NP, D = y.shape
    B = NP // 8

    def body(y_ref, h_ref, di_ref, b_ref, w_ref, o_ref):
        s = y_ref[0] + y_ref[1] + h_ref[...]
        z = jnp.maximum(di_ref[...] * s + b_ref[...], 0.0)
        o_ref[...] = di_ref[...] * jnp.dot(z, w_ref[...],
                                           preferred_element_type=jnp.float32)

    return pl.pallas_call(
        body,
        grid=(8,),
        in_specs=[pl.BlockSpec((2, B, D), lambda i: (0, i, 0)),
                  pl.BlockSpec((B, D), lambda i: (i, 0)),
                  pl.BlockSpec((B, D), lambda i: (i, 0)),
                  pl.BlockSpec((1, D), lambda i: (0, 0)),
                  pl.BlockSpec((D, D), lambda i: (0, 0))],
        out_specs=pl.BlockSpec((B, D), lambda i: (i, 0)),
        out_shape=jax.ShapeDtypeStruct((NP, D), jnp.float32),
    )(y, ht, dinvb, b, W)


def _final(y, ht, dinvb, b, N):
    _, NP, D = y.shape
    B = N // 10

    def body(y_ref, h_ref, di_ref, b_ref, o_ref):
        s = y_ref[0] + y_ref[1] + h_ref[...]
        o_ref[...] = di_ref[...] * s + b_ref[...]

    return pl.pallas_call(
        body,
        grid=(10,),
        in_specs=[pl.BlockSpec((2, B, D), lambda i: (0, i, 0)),
                  pl.BlockSpec((B, D), lambda i: (i, 0)),
                  pl.BlockSpec((B, D), lambda i: (i, 0)),
                  pl.BlockSpec((1, D), lambda i: (0, 0))],
        out_specs=pl.BlockSpec((B, D), lambda i: (i, 0)),
        out_shape=jax.ShapeDtypeStruct((N, D), jnp.float32),
    )(y, ht, dinvb, b)


def kernel(x, edge_index, W1, b1, W2, b2):
    N, D = x.shape
    E = edge_index.shape[1]
    NP = -(-(N + PAD_ROWS) // 1024) * 1024
    T = E // CHUNK

    ei = edge_index.astype(jnp.int32)
    sdx = ei.reshape(2, T, CHUNK).transpose(1, 0, 2)
    sdx64 = (sdx.reshape(T, 2, 2, SCH).transpose(0, 2, 1, 3)
             .reshape(2 * T, 2, SCH))

    xp = jnp.pad(x, ((0, NP - N), (0, 0)))
    zeros2 = jnp.zeros((NP, D), jnp.float32)

    deg_k = _make_deg_kernel(NP, T)
    scat_k = _make_scatter_kernel(NP, T, D)

    degf = deg_k(sdx).reshape(NW, NP // 128, 128)
    g1 = _matmul(xp, W1)
    dinvb, h1t = _prep(degf, g1)
    y1 = scat_k(h1t, sdx64, zeros2)
    h2t = _mid(y1, h1t, dinvb, b1.reshape(1, D), W2)
    y2 = scat_k(h2t, sdx64, zeros2)
    return _final(y2, h2t, dinvb, b2.reshape(1, D), N)
